# baseline (speedup 1.0000x reference)
import jax
import jax.numpy as jnp
from jax.experimental import pallas as pl
from jax.experimental.pallas import tpu as pltpu


def _sigmoid(x):
    return 0.5 * (jnp.tanh(0.5 * x) + 1.0)


def _round_up(v, m):
    return ((v + m - 1) // m) * m


def _tree_sum(xs):
    while len(xs) > 1:
        nxt = [xs[i] + xs[i + 1] for i in range(0, len(xs) - 1, 2)]
        if len(xs) % 2:
            nxt.append(xs[-1])
        xs = nxt
    return xs[0]


def _make_conv_body(TB, L, pad, K, C1, C2):
    Lp = L + 2 * pad

    def body(x2_ref, w1t_ref, b1_ref, w2t_ref, b2_ref, out_ref):
        f32 = jnp.float32
        bf16 = jnp.bfloat16
        x2 = x2_ref[...]

        h1 = _tree_sum([
            jnp.dot(w1t_ref[k], x2[:, k * TB:(k + L) * TB],
                    preferred_element_type=f32)
            for k in range(K)])
        h1 = jnp.maximum(h1 + b1_ref[...], 0.0).astype(bf16)
        zc1 = jnp.zeros((C1, pad * TB), bf16)
        h1p = jnp.concatenate([zc1, h1, zc1], axis=1)

        h2 = _tree_sum([
            jnp.dot(w2t_ref[k], h1p[:, k * TB:(k + L) * TB],
                    preferred_element_type=f32)
            for k in range(K)])
        h2 = jnp.maximum(h2 + b2_ref[...], 0.0)

        pool = _tree_sum([h2[:, l * TB:(l + 1) * TB] for l in range(L)])
        out_ref[...] = (pool * (1.0 / L)).reshape(1, C2, TB)

    return body


def _make_rnn_body(TB, S, L, pad, C8, H, C2, F1):
    Lp = L + 2 * pad
    TBs = TB // S

    def body(x8_ref, wcat_ref, cnn_ref, wfc1_ref, bfc1_ref,
             wfc2_ref, bfc2_ref, alpha_ref, out_ref):
        f32 = jnp.float32
        bf16 = jnp.bfloat16
        x8 = x8_ref[...]
        wcat = wcat_ref[...]
        ones_row = jnp.ones((8, TBs), bf16)

        hs = [jnp.zeros((H, TBs), f32) for _ in range(S)]
        cs = [jnp.zeros((H, TBs), f32) for _ in range(S)]
        for t in range(L):
            base = (t + pad) * TB
            for s in range(S):
                lo = base + s * TBs
                rhs = jnp.concatenate(
                    [x8[:, lo:lo + TBs], hs[s].astype(bf16), ones_row],
                    axis=0)
                g = jnp.dot(wcat, rhs, preferred_element_type=f32)
                s_if = _sigmoid(g[0:2 * H])
                s_o = _sigmoid(g[3 * H:4 * H])
                t_g = jnp.tanh(g[2 * H:3 * H])
                cs[s] = s_if[H:2 * H] * cs[s] + s_if[0:H] * t_g
                hs[s] = s_o * jnp.tanh(cs[s])
        rnn_t = jnp.concatenate(hs, axis=1)

        a = alpha_ref[0]
        fused = a * cnn_ref[0] + (1.0 - a) * rnn_t
        z1 = jnp.dot(wfc1_ref[...], fused.astype(bf16),
                     preferred_element_type=f32) + bfc1_ref[...]
        z1 = jnp.maximum(z1, 0.0)
        z2 = jnp.dot(wfc2_ref[...], z1.astype(bf16),
                     preferred_element_type=f32) + bfc2_ref[...]
        out_ref[...] = _sigmoid(z2).reshape(1, 1, TB)

    return body


def kernel(x, w1, b1, w2, b2, w_ih, w_hh, b_ih, b_hh,
           alpha, wfc1, bfc1, wfc2, bfc2):
    B, C_in, L = x.shape
    C1, _, K = w1.shape
    C2 = w2.shape[0]
    H = w_hh.shape[1]
    F1 = wfc1.shape[0]
    pad = K // 2
    Lp = L + 2 * pad

    TB = 4096 if B >= 8192 else max(128, _round_up(B, 128))
    S = max(1, TB // 128)
    B_pad = _round_up(B, TB)
    nt = B_pad // TB

    bf16 = jnp.bfloat16
    f32 = jnp.float32
    xp = jnp.pad(x.astype(bf16), ((0, B_pad - B), (0, 0), (pad, pad)))
    xp = xp.reshape(nt, TB, C_in, Lp)
    x2 = jnp.transpose(xp, (2, 0, 3, 1)).reshape(C_in, nt * Lp * TB)
    x8 = jnp.pad(xp, ((0, 0), (0, 0), (0, 8 - C_in), (0, 0)))
    x8 = jnp.transpose(x8, (2, 0, 3, 1)).reshape(8, nt * Lp * TB)

    w1t = jnp.transpose(w1, (2, 0, 1)).astype(bf16)
    w2t = jnp.transpose(w2, (2, 0, 1)).astype(bf16)
    b1c = b1.reshape(C1, 1)
    b2c = b2.reshape(C2, 1)
    wcat = jnp.concatenate([
        w_ih, jnp.zeros((4 * H, 8 - C_in), f32), w_hh,
        (b_ih + b_hh).reshape(4 * H, 1), jnp.zeros((4 * H, 7), f32),
    ], axis=1).astype(bf16)
    wfc1_b = wfc1.astype(bf16)
    bfc1c = bfc1.reshape(F1, 1)
    wfc2_b = wfc2.astype(bf16)
    bfc2c = bfc2.reshape(1, 1)
    alpha_s = jnp.asarray(alpha, f32).reshape(1)

    conv_body = _make_conv_body(TB, L, pad, K, C1, C2)
    cnn = pl.pallas_call(
        conv_body,
        out_shape=jax.ShapeDtypeStruct((nt, C2, TB), f32),
        grid_spec=pltpu.PrefetchScalarGridSpec(
            num_scalar_prefetch=0,
            grid=(nt,),
            in_specs=[
                pl.BlockSpec((C_in, Lp * TB), lambda i: (0, i)),
                pl.BlockSpec((K, C1, C_in), lambda i: (0, 0, 0)),
                pl.BlockSpec((C1, 1), lambda i: (0, 0)),
                pl.BlockSpec((K, C2, C1), lambda i: (0, 0, 0)),
                pl.BlockSpec((C2, 1), lambda i: (0, 0)),
            ],
            out_specs=pl.BlockSpec((1, C2, TB), lambda i: (i, 0, 0)),
        ),
        compiler_params=pltpu.CompilerParams(
            dimension_semantics=("parallel",),
            vmem_limit_bytes=56 * 1024 * 1024,
        ),
    )(x2, w1t, b1c, w2t, b2c)

    rnn_body = _make_rnn_body(TB, S, L, pad, 8, H, C2, F1)
    out = pl.pallas_call(
        rnn_body,
        out_shape=jax.ShapeDtypeStruct((nt, 1, TB), f32),
        grid_spec=pltpu.PrefetchScalarGridSpec(
            num_scalar_prefetch=0,
            grid=(nt,),
            in_specs=[
                pl.BlockSpec((8, Lp * TB), lambda i: (0, i)),
                pl.BlockSpec((4 * H, 48), lambda i: (0, 0)),
                pl.BlockSpec((1, C2, TB), lambda i: (i, 0, 0)),
                pl.BlockSpec((F1, C2), lambda i: (0, 0)),
                pl.BlockSpec((F1, 1), lambda i: (0, 0)),
                pl.BlockSpec((1, F1), lambda i: (0, 0)),
                pl.BlockSpec((1, 1), lambda i: (0, 0)),
                pl.BlockSpec(memory_space=pltpu.MemorySpace.SMEM),
            ],
            out_specs=pl.BlockSpec((1, 1, TB), lambda i: (i, 0, 0)),
        ),
        compiler_params=pltpu.CompilerParams(
            dimension_semantics=("parallel",),
            vmem_limit_bytes=45 * 1024 * 1024,
        ),
    )(x8, wcat, cnn, wfc1_b, bfc1c, wfc2_b, bfc2c, alpha_s)
    return out.reshape(B_pad, 1)[:B]


# speedup vs baseline: 1.0560x; 1.0560x over previous
import jax
import jax.numpy as jnp
from jax.experimental import pallas as pl
from jax.experimental.pallas import tpu as pltpu


def _sigmoid(x):
    return 0.5 * (jnp.tanh(0.5 * x) + 1.0)


def _round_up(v, m):
    return ((v + m - 1) // m) * m


def _tree_sum(xs):
    while len(xs) > 1:
        nxt = [xs[i] + xs[i + 1] for i in range(0, len(xs) - 1, 2)]
        if len(xs) % 2:
            nxt.append(xs[-1])
        xs = nxt
    return xs[0]


def _make_conv_body(TB, L, pad, K, C1, C2):
    Lp = L + 2 * pad

    def body(x2_ref, w1a_ref, w1b_ref, b1_ref, w2t_ref, b2_ref, out_ref):
        f32 = jnp.float32
        bf16 = jnp.bfloat16
        x2 = x2_ref[...]

        qa = jnp.dot(w1a_ref[...], x2[:, 0:(L + 3 - 1) * TB],
                     preferred_element_type=f32)
        qb = jnp.dot(w1b_ref[...], x2[:, 3 * TB:(L + 4) * TB],
                     preferred_element_type=f32)
        h1 = _tree_sum(
            [qa[r * C1:(r + 1) * C1, r * TB:(r + L) * TB] for r in range(3)] +
            [qb[r * C1:(r + 1) * C1, r * TB:(r + L) * TB] for r in range(2)])
        h1 = jnp.maximum(h1 + b1_ref[...], 0.0).astype(bf16)
        zc1 = jnp.zeros((C1, pad * TB), bf16)
        h1p = jnp.concatenate([zc1, h1, zc1], axis=1)

        h2 = _tree_sum([
            jnp.dot(w2t_ref[k], h1p[:, k * TB:(k + L) * TB],
                    preferred_element_type=f32)
            for k in range(K)])
        h2 = jnp.maximum(h2 + b2_ref[...], 0.0)

        pool = _tree_sum([h2[:, l * TB:(l + 1) * TB] for l in range(L)])
        out_ref[...] = (pool * (1.0 / L)).reshape(1, C2, TB)

    return body


def _make_rnn_body(TB, S, L, pad, C8, H, C2, F1):
    Lp = L + 2 * pad
    TBs = TB // S

    def body(x8_ref, wcat_ref, cnn_ref, wfc1_ref, bfc1_ref,
             wfc2_ref, bfc2_ref, alpha_ref, out_ref):
        f32 = jnp.float32
        bf16 = jnp.bfloat16
        x8 = x8_ref[...]
        wcat = wcat_ref[...]
        ones_row = jnp.ones((8, TBs), bf16)

        hs = [jnp.zeros((H, TBs), f32) for _ in range(S)]
        cs = [jnp.zeros((H, TBs), f32) for _ in range(S)]
        for t in range(L):
            base = (t + pad) * TB
            for s in range(S):
                lo = base + s * TBs
                rhs = jnp.concatenate(
                    [x8[:, lo:lo + TBs], hs[s].astype(bf16), ones_row],
                    axis=0)
                g = jnp.dot(wcat, rhs, preferred_element_type=f32)
                s_if = 0.5 * jnp.tanh(g[0:2 * H]) + 0.5
                s_o = 0.5 * jnp.tanh(g[3 * H:4 * H]) + 0.5
                t_g = jnp.tanh(g[2 * H:3 * H])
                cs[s] = s_if[H:2 * H] * cs[s] + s_if[0:H] * t_g
                hs[s] = s_o * jnp.tanh(cs[s])
        rnn_t = jnp.concatenate(hs, axis=1)

        a = alpha_ref[0]
        fused = a * cnn_ref[0] + (1.0 - a) * rnn_t
        z1 = jnp.dot(wfc1_ref[...], fused.astype(bf16),
                     preferred_element_type=f32) + bfc1_ref[...]
        z1 = jnp.maximum(z1, 0.0)
        z2 = jnp.dot(wfc2_ref[...], z1.astype(bf16),
                     preferred_element_type=f32) + bfc2_ref[...]
        out_ref[...] = _sigmoid(z2).reshape(1, 1, TB)

    return body


def kernel(x, w1, b1, w2, b2, w_ih, w_hh, b_ih, b_hh,
           alpha, wfc1, bfc1, wfc2, bfc2):
    B, C_in, L = x.shape
    C1, _, K = w1.shape
    C2 = w2.shape[0]
    H = w_hh.shape[1]
    F1 = wfc1.shape[0]
    pad = K // 2
    Lp = L + 2 * pad

    TB = 2048 if B >= 4096 else max(128, _round_up(B, 128))
    S = max(1, TB // 128)
    B_pad = _round_up(B, TB)
    nt = B_pad // TB

    bf16 = jnp.bfloat16
    f32 = jnp.float32
    xp = jnp.pad(x.astype(bf16), ((0, B_pad - B), (0, 0), (pad, pad)))
    xp = xp.reshape(nt, TB, C_in, Lp)
    x2 = jnp.transpose(xp, (2, 0, 3, 1)).reshape(C_in, nt * Lp * TB)
    x8 = jnp.pad(xp, ((0, 0), (0, 0), (0, 8 - C_in), (0, 0)))
    x8 = jnp.transpose(x8, (2, 0, 3, 1)).reshape(8, nt * Lp * TB)

    w1t = jnp.transpose(w1, (2, 0, 1))
    w1a = w1t[0:3].reshape(3 * C1, C_in).astype(bf16)
    w1b = w1t[3:5].reshape(2 * C1, C_in).astype(bf16)
    w2t = jnp.transpose(w2, (2, 0, 1)).astype(bf16)
    b1c = b1.reshape(C1, 1)
    b2c = b2.reshape(C2, 1)
    wcat = jnp.concatenate([
        w_ih, jnp.zeros((4 * H, 8 - C_in), f32), w_hh,
        (b_ih + b_hh).reshape(4 * H, 1), jnp.zeros((4 * H, 7), f32),
    ], axis=1)
    gate_scale = jnp.concatenate([
        jnp.full((2 * H, 1), 0.5, f32), jnp.ones((H, 1), f32),
        jnp.full((H, 1), 0.5, f32)], axis=0)
    wcat = (wcat * gate_scale).astype(bf16)
    wfc1_b = wfc1.astype(bf16)
    bfc1c = bfc1.reshape(F1, 1)
    wfc2_b = wfc2.astype(bf16)
    bfc2c = bfc2.reshape(1, 1)
    alpha_s = jnp.asarray(alpha, f32).reshape(1)

    conv_body = _make_conv_body(TB, L, pad, K, C1, C2)
    cnn = pl.pallas_call(
        conv_body,
        out_shape=jax.ShapeDtypeStruct((nt, C2, TB), f32),
        grid_spec=pltpu.PrefetchScalarGridSpec(
            num_scalar_prefetch=0,
            grid=(nt,),
            in_specs=[
                pl.BlockSpec((C_in, Lp * TB), lambda i: (0, i)),
                pl.BlockSpec((3 * C1, C_in), lambda i: (0, 0)),
                pl.BlockSpec((2 * C1, C_in), lambda i: (0, 0)),
                pl.BlockSpec((C1, 1), lambda i: (0, 0)),
                pl.BlockSpec((K, C2, C1), lambda i: (0, 0, 0)),
                pl.BlockSpec((C2, 1), lambda i: (0, 0)),
            ],
            out_specs=pl.BlockSpec((1, C2, TB), lambda i: (i, 0, 0)),
        ),
        compiler_params=pltpu.CompilerParams(
            dimension_semantics=("parallel",),
            vmem_limit_bytes=56 * 1024 * 1024,
        ),
    )(x2, w1a, w1b, b1c, w2t, b2c)

    rnn_body = _make_rnn_body(TB, S, L, pad, 8, H, C2, F1)
    out = pl.pallas_call(
        rnn_body,
        out_shape=jax.ShapeDtypeStruct((nt, 1, TB), f32),
        grid_spec=pltpu.PrefetchScalarGridSpec(
            num_scalar_prefetch=0,
            grid=(nt,),
            in_specs=[
                pl.BlockSpec((8, Lp * TB), lambda i: (0, i)),
                pl.BlockSpec((4 * H, 48), lambda i: (0, 0)),
                pl.BlockSpec((1, C2, TB), lambda i: (i, 0, 0)),
                pl.BlockSpec((F1, C2), lambda i: (0, 0)),
                pl.BlockSpec((F1, 1), lambda i: (0, 0)),
                pl.BlockSpec((1, F1), lambda i: (0, 0)),
                pl.BlockSpec((1, 1), lambda i: (0, 0)),
                pl.BlockSpec(memory_space=pltpu.MemorySpace.SMEM),
            ],
            out_specs=pl.BlockSpec((1, 1, TB), lambda i: (i, 0, 0)),
        ),
        compiler_params=pltpu.CompilerParams(
            dimension_semantics=("parallel",),
            vmem_limit_bytes=45 * 1024 * 1024,
        ),
    )(x8, wcat, cnn, wfc1_b, bfc1c, wfc2_b, bfc2c, alpha_s)
    return out.reshape(B_pad, 1)[:B]


# speedup vs baseline: 1.0595x; 1.0033x over previous
import jax
import jax.numpy as jnp
from jax.experimental import pallas as pl
from jax.experimental.pallas import tpu as pltpu


def _sigmoid(x):
    return 0.5 * (jnp.tanh(0.5 * x) + 1.0)


def _round_up(v, m):
    return ((v + m - 1) // m) * m


def _tree_sum(xs):
    while len(xs) > 1:
        nxt = [xs[i] + xs[i + 1] for i in range(0, len(xs) - 1, 2)]
        if len(xs) % 2:
            nxt.append(xs[-1])
        xs = nxt
    return xs[0]


def _make_conv_body(TB, L, pad, K, C1, C2):
    Lp = L + 2 * pad

    def body(x2_ref, w1t_ref, b1_ref, w2t_ref, b2_ref, out_ref):
        f32 = jnp.float32
        bf16 = jnp.bfloat16
        x2 = x2_ref[...]

        h1 = _tree_sum([
            jnp.dot(w1t_ref[k], x2[:, k * TB:(k + L) * TB],
                    preferred_element_type=f32)
            for k in range(K)])
        h1 = jnp.maximum(h1 + b1_ref[...], 0.0).astype(bf16)
        zc1 = jnp.zeros((C1, pad * TB), bf16)
        h1p = jnp.concatenate([zc1, h1, zc1], axis=1)

        h2 = _tree_sum([
            jnp.dot(w2t_ref[k], h1p[:, k * TB:(k + L) * TB],
                    preferred_element_type=f32)
            for k in range(K)])
        h2 = jnp.maximum(h2 + b2_ref[...], 0.0)

        pool = _tree_sum([h2[:, l * TB:(l + 1) * TB] for l in range(L)])
        out_ref[...] = (pool * (1.0 / L)).reshape(1, C2, TB)

    return body


def _make_rnn_body(TB, S, L, pad, C8, H, C2, F1):
    Lp = L + 2 * pad
    TBs = TB // S

    def body(x8_ref, wcat_ref, cnn_ref, wfc1_ref, bfc1_ref,
             wfc2_ref, bfc2_ref, alpha_ref, out_ref):
        f32 = jnp.float32
        bf16 = jnp.bfloat16
        x8 = x8_ref[...]
        wcat = wcat_ref[...]
        ones_row = jnp.ones((8, TBs), bf16)

        hs = [jnp.zeros((H, TBs), f32) for _ in range(S)]
        cs = [jnp.zeros((H, TBs), f32) for _ in range(S)]
        for t in range(L):
            base = (t + pad) * TB
            for s in range(S):
                lo = base + s * TBs
                rhs = jnp.concatenate(
                    [x8[:, lo:lo + TBs], hs[s].astype(bf16), ones_row],
                    axis=0)
                g = jnp.dot(wcat, rhs, preferred_element_type=f32)
                s_if = _sigmoid(g[0:2 * H])
                s_o = _sigmoid(g[3 * H:4 * H])
                t_g = jnp.tanh(g[2 * H:3 * H])
                cs[s] = s_if[H:2 * H] * cs[s] + s_if[0:H] * t_g
                hs[s] = s_o * jnp.tanh(cs[s])
        rnn_t = jnp.concatenate(hs, axis=1)

        a = alpha_ref[0]
        fused = a * cnn_ref[0] + (1.0 - a) * rnn_t
        z1 = jnp.dot(wfc1_ref[...], fused.astype(bf16),
                     preferred_element_type=f32) + bfc1_ref[...]
        z1 = jnp.maximum(z1, 0.0)
        z2 = jnp.dot(wfc2_ref[...], z1.astype(bf16),
                     preferred_element_type=f32) + bfc2_ref[...]
        out_ref[...] = _sigmoid(z2).reshape(1, 1, TB)

    return body


def kernel(x, w1, b1, w2, b2, w_ih, w_hh, b_ih, b_hh,
           alpha, wfc1, bfc1, wfc2, bfc2):
    B, C_in, L = x.shape
    C1, _, K = w1.shape
    C2 = w2.shape[0]
    H = w_hh.shape[1]
    F1 = wfc1.shape[0]
    pad = K // 2
    Lp = L + 2 * pad

    TB = 2048 if B >= 4096 else max(128, _round_up(B, 128))
    S = max(1, TB // 128)
    B_pad = _round_up(B, TB)
    nt = B_pad // TB

    bf16 = jnp.bfloat16
    f32 = jnp.float32
    xp = jnp.pad(x.astype(bf16), ((0, B_pad - B), (0, 0), (pad, pad)))
    xp = xp.reshape(nt, TB, C_in, Lp)
    x2 = jnp.transpose(xp, (2, 0, 3, 1)).reshape(C_in, nt * Lp * TB)
    x8 = jnp.pad(xp, ((0, 0), (0, 0), (0, 8 - C_in), (0, 0)))
    x8 = jnp.transpose(x8, (2, 0, 3, 1)).reshape(8, nt * Lp * TB)

    w1t = jnp.transpose(w1, (2, 0, 1)).astype(bf16)
    w2t = jnp.transpose(w2, (2, 0, 1)).astype(bf16)
    b1c = b1.reshape(C1, 1)
    b2c = b2.reshape(C2, 1)
    wcat = jnp.concatenate([
        w_ih, jnp.zeros((4 * H, 8 - C_in), f32), w_hh,
        (b_ih + b_hh).reshape(4 * H, 1), jnp.zeros((4 * H, 7), f32),
    ], axis=1).astype(bf16)
    wfc1_b = wfc1.astype(bf16)
    bfc1c = bfc1.reshape(F1, 1)
    wfc2_b = wfc2.astype(bf16)
    bfc2c = bfc2.reshape(1, 1)
    alpha_s = jnp.asarray(alpha, f32).reshape(1)

    conv_body = _make_conv_body(TB, L, pad, K, C1, C2)
    cnn = pl.pallas_call(
        conv_body,
        out_shape=jax.ShapeDtypeStruct((nt, C2, TB), f32),
        grid_spec=pltpu.PrefetchScalarGridSpec(
            num_scalar_prefetch=0,
            grid=(nt,),
            in_specs=[
                pl.BlockSpec((C_in, Lp * TB), lambda i: (0, i)),
                pl.BlockSpec((K, C1, C_in), lambda i: (0, 0, 0)),
                pl.BlockSpec((C1, 1), lambda i: (0, 0)),
                pl.BlockSpec((K, C2, C1), lambda i: (0, 0, 0)),
                pl.BlockSpec((C2, 1), lambda i: (0, 0)),
            ],
            out_specs=pl.BlockSpec((1, C2, TB), lambda i: (i, 0, 0)),
        ),
        compiler_params=pltpu.CompilerParams(
            dimension_semantics=("parallel",),
            vmem_limit_bytes=56 * 1024 * 1024,
        ),
    )(x2, w1t, b1c, w2t, b2c)

    rnn_body = _make_rnn_body(TB, S, L, pad, 8, H, C2, F1)
    out = pl.pallas_call(
        rnn_body,
        out_shape=jax.ShapeDtypeStruct((nt, 1, TB), f32),
        grid_spec=pltpu.PrefetchScalarGridSpec(
            num_scalar_prefetch=0,
            grid=(nt,),
            in_specs=[
                pl.BlockSpec((8, Lp * TB), lambda i: (0, i)),
                pl.BlockSpec((4 * H, 48), lambda i: (0, 0)),
                pl.BlockSpec((1, C2, TB), lambda i: (i, 0, 0)),
                pl.BlockSpec((F1, C2), lambda i: (0, 0)),
                pl.BlockSpec((F1, 1), lambda i: (0, 0)),
                pl.BlockSpec((1, F1), lambda i: (0, 0)),
                pl.BlockSpec((1, 1), lambda i: (0, 0)),
                pl.BlockSpec(memory_space=pltpu.MemorySpace.SMEM),
            ],
            out_specs=pl.BlockSpec((1, 1, TB), lambda i: (i, 0, 0)),
        ),
        compiler_params=pltpu.CompilerParams(
            dimension_semantics=("parallel",),
            vmem_limit_bytes=45 * 1024 * 1024,
        ),
    )(x8, wcat, cnn, wfc1_b, bfc1c, wfc2_b, bfc2c, alpha_s)
    return out.reshape(B_pad, 1)[:B]
